# Initial kernel scaffold; baseline (speedup 1.0000x reference)
import jax
import jax.numpy as jnp
from jax.experimental import pallas as pl


def kernel(annotation, segment_ids, W_reduce, b_reduce, W_gate, b_gate, W_out, b_out):
    raise NotImplementedError("write your pallas kernel here")


# speedup vs baseline: 14.3032x; 14.3032x over previous
import jax
import jax.numpy as jnp
from jax.experimental import pallas as pl
from jax.experimental.pallas import tpu as pltpu

_N = 100000
_B = 256
_ANN = 128
_H = 64
_BN = 4000
_GRID = _N // _BN


def _pool_kernel(segf_ref, ann_ref, wred_ref, bred_ref, wg_ref, wo_ref,
                 bg_ref, bo_ref, out_ref, m_ref, d_ref, t_ref):
    i = pl.program_id(0)

    @pl.when(i == 0)
    def _init():
        m_ref[...] = jnp.full((1, _B), -jnp.inf, jnp.float32)
        d_ref[...] = jnp.zeros((1, _B), jnp.float32)
        t_ref[...] = jnp.zeros((1, _B), jnp.float32)

    wg2 = wg_ref[0:_H, :] + wg_ref[_H:2 * _H, :]
    wo2 = wo_ref[0:_H, :] + wo_ref[_H:2 * _H, :]
    w2 = jnp.dot(wred_ref[...], jnp.concatenate([wg2, wo2], axis=1),
                 preferred_element_type=jnp.float32)
    st = jnp.dot(ann_ref[...], w2, preferred_element_type=jnp.float32)
    s = st[:, 0:1]
    t = st[:, 1:2]

    seg_iota = jax.lax.broadcasted_iota(jnp.int32, (_BN, _B), 1).astype(jnp.float32)
    onehot = segf_ref[...] == seg_iota
    oh_f = onehot.astype(jnp.float32)

    lm = jnp.max(jnp.where(onehot, jnp.broadcast_to(s, (_BN, _B)), -jnp.inf),
                 axis=0, keepdims=True)
    m_old = m_ref[...]
    m_new = jnp.maximum(m_old, lm)
    scale = jnp.where(m_old == -jnp.inf, 0.0, jnp.exp(m_old - m_new))

    m_g = jnp.where(m_new == -jnp.inf, 0.0, m_new)
    m_row = jnp.dot(oh_f, m_g.reshape(_B, 1),
                    preferred_element_type=jnp.float32)
    e = jnp.exp(s - m_row)
    ew = jnp.where(onehot, jnp.broadcast_to(e, (_BN, _B)), 0.0)
    d_part = jnp.sum(ew, axis=0, keepdims=True)
    t_part = jnp.sum(ew * t, axis=0, keepdims=True)

    m_ref[...] = m_new
    d_ref[...] = d_ref[...] * scale + d_part
    t_ref[...] = t_ref[...] * scale + t_part

    @pl.when(i == _GRID - 1)
    def _finish():
        c1 = jnp.dot(bred_ref[...], wo2,
                     preferred_element_type=jnp.float32)
        dv = d_ref[...]
        out_ref[...] = (t_ref[...] / jnp.maximum(dv, 1e-12)
                        + jnp.where(dv > 0, 1.0, 0.0) * c1[0, 0]
                        + bo_ref[0, 0])


def kernel(annotation, segment_ids, W_reduce, b_reduce, W_gate, b_gate,
           W_out, b_out):
    segf = segment_ids.astype(jnp.float32).reshape(_N, 1)
    out = pl.pallas_call(
        _pool_kernel,
        grid=(_GRID,),
        in_specs=[
            pl.BlockSpec((_BN, 1), lambda i: (i, 0)),
            pl.BlockSpec((_BN, _ANN), lambda i: (i, 0)),
            pl.BlockSpec((_ANN, _H), lambda i: (0, 0)),
            pl.BlockSpec((1, _H), lambda i: (0, 0)),
            pl.BlockSpec((2 * _H, 1), lambda i: (0, 0)),
            pl.BlockSpec((2 * _H, 1), lambda i: (0, 0)),
            pl.BlockSpec((1, 1), lambda i: (0, 0)),
            pl.BlockSpec((1, 1), lambda i: (0, 0)),
        ],
        out_specs=pl.BlockSpec((1, _B), lambda i: (0, 0)),
        out_shape=jax.ShapeDtypeStruct((1, _B), jnp.float32),
        scratch_shapes=[
            pltpu.VMEM((1, _B), jnp.float32),
            pltpu.VMEM((1, _B), jnp.float32),
            pltpu.VMEM((1, _B), jnp.float32),
        ],
    )(segf, annotation, W_reduce, b_reduce.reshape(1, _H), W_gate, W_out,
      b_gate.reshape(1, 1), b_out.reshape(1, 1))
    return out.reshape(_B)
